# Initial kernel scaffold; baseline (speedup 1.0000x reference)
import jax
import jax.numpy as jnp
from jax.experimental import pallas as pl


def kernel(x, edge_index, W1, b1, W2, b2):
    raise NotImplementedError("write your pallas kernel here")


# speedup vs baseline: 32.1723x; 32.1723x over previous
import functools

import jax
import jax.numpy as jnp
from jax import lax
from jax.experimental import pallas as pl
from jax.experimental.pallas import tpu as pltpu
from jax.experimental.pallas import tpu_sc as plsc

N = 10000
E = 320000
IN_CH = 128
HID = 16
OUT_CH = 128

NC = 2
NS = 16
NW = NC * NS
L = 16

EPW = E // NW
B = 128
C = (EPW + B - 1) // B
EPW_P = C * B
RPT = 632
NPAD = NS * RPT

_MESH = plsc.VectorSubcoreMesh(core_axis_name="c", subcore_axis_name="s")
_SC_PARAMS = pltpu.CompilerParams(use_tc_tiling_on_sc=False)


def _zero_slice_and_barrier(stage_v, acc_sh, s):
    def zfill(i, _):
        stage_v[i, :] = jnp.zeros((L,), jnp.float32)
        return 0
    lax.fori_loop(0, RPT, zfill, 0)
    pltpu.sync_copy(stage_v, acc_sh.at[pl.ds(s * RPT, RPT)])
    plsc.subcore_barrier()


def _writeback(stage_v, acc_sh, out_hbm, c, s):
    plsc.subcore_barrier()
    pltpu.sync_copy(acc_sh.at[pl.ds(s * RPT, RPT)], stage_v)
    pltpu.sync_copy(stage_v, out_hbm.at[c, pl.ds(s * RPT, RPT)])


@functools.partial(
    pl.kernel,
    out_type=jax.ShapeDtypeStruct((NC, NPAD, L), jnp.float32),
    mesh=_MESH,
    scratch_types=[
        pltpu.VMEM((C, B), jnp.int32),
        pltpu.VMEM((B, L), jnp.float32),
        pltpu.VMEM((RPT, L), jnp.float32),
        pltpu.VMEM_SHARED((NPAD, L), jnp.float32),
    ],
    compiler_params=_SC_PARAMS,
)
def _deg_kernel(dst_hbm, out_hbm, dst_v, ones_v, stage_v, acc_sh):
    c = lax.axis_index("c")
    s = lax.axis_index("s")
    w = c * NS + s

    def ofill(i, _):
        ones_v[i, :] = jnp.full((L,), 1.0, jnp.float32)
        return 0
    lax.fori_loop(0, B, ofill, 0)
    _zero_slice_and_barrier(stage_v, acc_sh, s)

    pltpu.sync_copy(dst_hbm.at[w], dst_v)

    def chunk(j, _):
        pltpu.sync_copy(ones_v, acc_sh.at[dst_v.at[j]], add=True)
        return 0
    lax.fori_loop(0, C, chunk, 0)

    _writeback(stage_v, acc_sh, out_hbm, c, s)


@functools.partial(
    pl.kernel,
    out_type=jax.ShapeDtypeStruct((NC, NPAD, L), jnp.float32),
    mesh=_MESH,
    scratch_types=[
        pltpu.VMEM((C, B), jnp.int32),
        pltpu.VMEM((C, B), jnp.int32),
        pltpu.VMEM((B, L), jnp.float32),
        pltpu.VMEM((RPT, L), jnp.float32),
        pltpu.VMEM_SHARED((NPAD, L), jnp.float32),
        pltpu.SemaphoreType.DMA,
    ],
    compiler_params=_SC_PARAMS,
)
def _agg_kernel(g_hbm, src_hbm, dst_hbm, out_hbm,
                src_v, dst_v, rows_v, stage_v, acc_sh, sem):
    c = lax.axis_index("c")
    s = lax.axis_index("s")
    w = c * NS + s

    _zero_slice_and_barrier(stage_v, acc_sh, s)

    pltpu.sync_copy(src_hbm.at[w], src_v)
    pltpu.sync_copy(dst_hbm.at[w], dst_v)

    def chunk(j, _):
        pltpu.async_copy(g_hbm.at[src_v.at[j]], rows_v, sem).wait()
        pltpu.sync_copy(rows_v, acc_sh.at[dst_v.at[j]], add=True)
        return 0
    lax.fori_loop(0, C, chunk, 0)

    _writeback(stage_v, acc_sh, out_hbm, c, s)


def _tc1_body(degp_ref, x_ref, w1_ref, g1_ref, dinv_ref):
    deg = degp_ref[0] + degp_ref[1] + 1.0
    dinv = lax.rsqrt(deg)
    h = jnp.dot(x_ref[:], w1_ref[:], preferred_element_type=jnp.float32)
    g1_ref[:] = h * dinv
    dinv_ref[:] = dinv


def _tc2_body(accp_ref, g1_ref, dinv_ref, b1_ref, g2_ref):
    dinv = dinv_ref[:]
    z = dinv * (accp_ref[0] + accp_ref[1] + g1_ref[:]) + b1_ref[:]
    g2_ref[:] = jnp.maximum(z, 0.0) * dinv


def _tc3_body(accp_ref, g2_ref, dinv_ref, w2_ref, b2_ref, out_ref):
    z = dinv_ref[:] * (accp_ref[0] + accp_ref[1] + g2_ref[:])
    out_ref[:] = (
        jnp.dot(z, w2_ref[:], preferred_element_type=jnp.float32) + b2_ref[:]
    )


def kernel(x, edge_index, W1, b1, W2, b2):
    src = edge_index[0].reshape(NW, EPW)
    dst = edge_index[1].reshape(NW, EPW)
    src_p = jnp.pad(src, ((0, 0), (0, EPW_P - EPW)),
                    constant_values=N).reshape(NW, C, B)
    dst_p = jnp.pad(dst, ((0, 0), (0, EPW_P - EPW)),
                    constant_values=N).reshape(NW, C, B)
    x_p = jnp.pad(x, ((0, NPAD - N), (0, 0)))
    b1r = b1.reshape(1, HID)
    b2r = b2.reshape(1, OUT_CH)

    deg_parts = _deg_kernel(dst_p)

    g1, dinv16 = pl.pallas_call(
        _tc1_body,
        out_shape=(
            jax.ShapeDtypeStruct((NPAD, HID), jnp.float32),
            jax.ShapeDtypeStruct((NPAD, HID), jnp.float32),
        ),
    )(deg_parts, x_p, W1)

    acc1 = _agg_kernel(g1, src_p, dst_p)

    g2 = pl.pallas_call(
        _tc2_body,
        out_shape=jax.ShapeDtypeStruct((NPAD, HID), jnp.float32),
    )(acc1, g1, dinv16, b1r)

    acc2 = _agg_kernel(g2, src_p, dst_p)

    out = pl.pallas_call(
        _tc3_body,
        out_shape=jax.ShapeDtypeStruct((NPAD, OUT_CH), jnp.float32),
    )(acc2, g2, dinv16, W2, b2r)

    return out[:N]


# speedup vs baseline: 39.2116x; 1.2188x over previous
import functools

import jax
import jax.numpy as jnp
from jax import lax
from jax.experimental import pallas as pl
from jax.experimental.pallas import tpu as pltpu
from jax.experimental.pallas import tpu_sc as plsc

N = 10000
E = 320000
IN_CH = 128
HID = 16
OUT_CH = 128

NC = 2
NS = 16
NW = NC * NS
L = 16

EPW = E // NW
B = 128
NBUF = 4
C = 80
EPW_P = C * B
G = C // NBUF
RPT = 632
NPAD = NS * RPT

_MESH = plsc.VectorSubcoreMesh(core_axis_name="c", subcore_axis_name="s")
_SC_PARAMS = pltpu.CompilerParams(use_tc_tiling_on_sc=False)


def _zero_slice_and_barrier(stage_v, acc_sh, s):
    def zfill(i, _):
        stage_v[i, :] = jnp.zeros((L,), jnp.float32)
        return 0
    lax.fori_loop(0, RPT, zfill, 0)
    pltpu.sync_copy(stage_v, acc_sh.at[pl.ds(s * RPT, RPT)])
    plsc.subcore_barrier()


def _writeback(stage_v, acc_sh, out_hbm, c, s):
    plsc.subcore_barrier()
    pltpu.sync_copy(acc_sh.at[pl.ds(s * RPT, RPT)], stage_v)
    pltpu.sync_copy(stage_v, out_hbm.at[c, pl.ds(s * RPT, RPT)])


@functools.partial(
    pl.kernel,
    out_type=jax.ShapeDtypeStruct((NC, NPAD, L), jnp.float32),
    mesh=_MESH,
    scratch_types=[
        pltpu.VMEM((C, B), jnp.int32),
        pltpu.VMEM((B, L), jnp.float32),
        pltpu.VMEM((RPT, L), jnp.float32),
        pltpu.VMEM_SHARED((NPAD, L), jnp.float32),
        pltpu.SemaphoreType.DMA,
    ],
    compiler_params=_SC_PARAMS,
)
def _deg_kernel(dst_hbm, out_hbm, dst_v, ones_v, stage_v, acc_sh, sem):
    c = lax.axis_index("c")
    s = lax.axis_index("s")
    w = c * NS + s

    def ofill(i, _):
        ones_v[i, :] = jnp.full((L,), 1.0, jnp.float32)
        return 0
    lax.fori_loop(0, B, ofill, 0)
    _zero_slice_and_barrier(stage_v, acc_sh, s)

    pltpu.sync_copy(dst_hbm.at[w], dst_v)

    def fire(j, _):
        pltpu.async_copy(ones_v, acc_sh.at[dst_v.at[j]], sem, add=True)
        return 0
    lax.fori_loop(0, C, fire, 0)

    def drain(j, _):
        pltpu.make_async_copy(ones_v, acc_sh.at[dst_v.at[j]], sem).wait()
        return 0
    lax.fori_loop(0, C, drain, 0)

    _writeback(stage_v, acc_sh, out_hbm, c, s)


@functools.partial(
    pl.kernel,
    out_type=jax.ShapeDtypeStruct((NC, NPAD, L), jnp.float32),
    mesh=_MESH,
    scratch_types=[
        pltpu.VMEM((C, B), jnp.int32),
        pltpu.VMEM((C, B), jnp.int32),
        pltpu.VMEM((NBUF, B, L), jnp.float32),
        pltpu.VMEM((RPT, L), jnp.float32),
        pltpu.VMEM_SHARED((NPAD, L), jnp.float32),
        pltpu.SemaphoreType.DMA((NBUF,)),
        pltpu.SemaphoreType.DMA((NBUF,)),
    ],
    compiler_params=_SC_PARAMS,
)
def _agg_kernel(g_hbm, src_hbm, dst_hbm, out_hbm,
                src_v, dst_v, rows_v, stage_v, acc_sh, gsem, ssem):
    c = lax.axis_index("c")
    s = lax.axis_index("s")
    w = c * NS + s

    _zero_slice_and_barrier(stage_v, acc_sh, s)

    pltpu.sync_copy(src_hbm.at[w], src_v)
    pltpu.sync_copy(dst_hbm.at[w], dst_v)

    def g_desc(j, b):
        return pltpu.make_async_copy(
            g_hbm.at[src_v.at[j]], rows_v.at[b], gsem.at[b])

    def s_desc(j, b):
        return pltpu.make_async_copy(
            rows_v.at[b], acc_sh.at[dst_v.at[j]], ssem.at[b])

    for b in range(NBUF):
        pltpu.async_copy(g_hbm.at[src_v.at[b]], rows_v.at[b], gsem.at[b])

    def group(i, _):
        for b in range(NBUF):
            j = i * NBUF + b
            g_desc(j, b).wait()
            pltpu.async_copy(rows_v.at[b], acc_sh.at[dst_v.at[j]],
                             ssem.at[b], add=True)
        for b in range(NBUF):
            j = i * NBUF + b
            s_desc(j, b).wait()
            jn = j + NBUF
            pltpu.async_copy(g_hbm.at[src_v.at[jn]], rows_v.at[b],
                             gsem.at[b])
        return 0
    lax.fori_loop(0, G - 1, group, 0)

    for b in range(NBUF):
        j = (G - 1) * NBUF + b
        g_desc(j, b).wait()
        pltpu.async_copy(rows_v.at[b], acc_sh.at[dst_v.at[j]],
                         ssem.at[b], add=True)
    for b in range(NBUF):
        s_desc((G - 1) * NBUF + b, b).wait()

    _writeback(stage_v, acc_sh, out_hbm, c, s)


def _tc1_body(degp_ref, x_ref, w1_ref, g1_ref, dinv_ref):
    deg = degp_ref[0] + degp_ref[1] + 1.0
    dinv = lax.rsqrt(deg)
    h = jnp.dot(x_ref[:], w1_ref[:], preferred_element_type=jnp.float32)
    g1_ref[:] = h * dinv
    dinv_ref[:] = dinv


def _tc2_body(accp_ref, g1_ref, dinv_ref, b1_ref, g2_ref):
    dinv = dinv_ref[:]
    z = dinv * (accp_ref[0] + accp_ref[1] + g1_ref[:]) + b1_ref[:]
    g2_ref[:] = jnp.maximum(z, 0.0) * dinv


def _tc3_body(accp_ref, g2_ref, dinv_ref, w2_ref, b2_ref, out_ref):
    z = dinv_ref[:] * (accp_ref[0] + accp_ref[1] + g2_ref[:])
    out_ref[:] = (
        jnp.dot(z, w2_ref[:], preferred_element_type=jnp.float32) + b2_ref[:]
    )


def kernel(x, edge_index, W1, b1, W2, b2):
    src = edge_index[0].reshape(NW, EPW)
    dst = edge_index[1].reshape(NW, EPW)
    src_p = jnp.pad(src, ((0, 0), (0, EPW_P - EPW)),
                    constant_values=N).reshape(NW, C, B)
    dst_p = jnp.pad(dst, ((0, 0), (0, EPW_P - EPW)),
                    constant_values=N).reshape(NW, C, B)
    x_p = jnp.pad(x, ((0, NPAD - N), (0, 0)))
    b1r = b1.reshape(1, HID)
    b2r = b2.reshape(1, OUT_CH)

    deg_parts = _deg_kernel(dst_p)

    g1, dinv16 = pl.pallas_call(
        _tc1_body,
        out_shape=(
            jax.ShapeDtypeStruct((NPAD, HID), jnp.float32),
            jax.ShapeDtypeStruct((NPAD, HID), jnp.float32),
        ),
    )(deg_parts, x_p, W1)

    acc1 = _agg_kernel(g1, src_p, dst_p)

    g2 = pl.pallas_call(
        _tc2_body,
        out_shape=jax.ShapeDtypeStruct((NPAD, HID), jnp.float32),
    )(acc1, g1, dinv16, b1r)

    acc2 = _agg_kernel(g2, src_p, dst_p)

    out = pl.pallas_call(
        _tc3_body,
        out_shape=jax.ShapeDtypeStruct((NPAD, OUT_CH), jnp.float32),
    )(acc2, g2, dinv16, W2, b2r)

    return out[:N]


# speedup vs baseline: 40.8968x; 1.0430x over previous
import functools

import jax
import jax.numpy as jnp
from jax import lax
from jax.experimental import pallas as pl
from jax.experimental.pallas import tpu as pltpu
from jax.experimental.pallas import tpu_sc as plsc

N = 10000
E = 320000
IN_CH = 128
HID = 16
OUT_CH = 128

NC = 2
NS = 16
NW = NC * NS
L = 16

EPW = E // NW
B = 128
NBUF = 8
C = 80
EPW_P = C * B
G = C // NBUF
RPT = 632
NPAD = NS * RPT

_MESH = plsc.VectorSubcoreMesh(core_axis_name="c", subcore_axis_name="s")
_SC_PARAMS = pltpu.CompilerParams(use_tc_tiling_on_sc=False)


def _zero_slice_and_barrier(stage_v, acc_sh, s):
    def zfill(i, _):
        stage_v[i, :] = jnp.zeros((L,), jnp.float32)
        return 0
    lax.fori_loop(0, RPT, zfill, 0)
    pltpu.sync_copy(stage_v, acc_sh.at[pl.ds(s * RPT, RPT)])
    plsc.subcore_barrier()


def _writeback(stage_v, acc_sh, out_hbm, c, s):
    plsc.subcore_barrier()
    pltpu.sync_copy(acc_sh.at[pl.ds(s * RPT, RPT)],
                    out_hbm.at[c, pl.ds(s * RPT, RPT)])


@functools.partial(
    pl.kernel,
    out_type=jax.ShapeDtypeStruct((NC, NPAD, L), jnp.float32),
    mesh=_MESH,
    scratch_types=[
        pltpu.VMEM((C, B), jnp.int32),
        pltpu.VMEM((B, L), jnp.float32),
        pltpu.VMEM((RPT, L), jnp.float32),
        pltpu.VMEM_SHARED((NPAD, L), jnp.float32),
        pltpu.SemaphoreType.DMA,
    ],
    compiler_params=_SC_PARAMS,
)
def _deg_kernel(dst_hbm, out_hbm, dst_v, ones_v, stage_v, acc_sh, sem):
    c = lax.axis_index("c")
    s = lax.axis_index("s")
    w = c * NS + s

    def ofill(i, _):
        ones_v[i, :] = jnp.full((L,), 1.0, jnp.float32)
        return 0
    lax.fori_loop(0, B, ofill, 0)
    _zero_slice_and_barrier(stage_v, acc_sh, s)

    pltpu.sync_copy(dst_hbm.at[w], dst_v)

    def fire(j, _):
        pltpu.async_copy(ones_v, acc_sh.at[dst_v.at[j]], sem, add=True)
        return 0
    lax.fori_loop(0, C, fire, 0)

    def drain(j, _):
        pltpu.make_async_copy(ones_v, acc_sh.at[dst_v.at[j]], sem).wait()
        return 0
    lax.fori_loop(0, C, drain, 0)

    _writeback(stage_v, acc_sh, out_hbm, c, s)


@functools.partial(
    pl.kernel,
    out_type=jax.ShapeDtypeStruct((NC, NPAD, L), jnp.float32),
    mesh=_MESH,
    scratch_types=[
        pltpu.VMEM((C, B), jnp.int32),
        pltpu.VMEM((C, B), jnp.int32),
        pltpu.VMEM((NBUF, B, L), jnp.float32),
        pltpu.VMEM((RPT, L), jnp.float32),
        pltpu.VMEM_SHARED((NPAD, L), jnp.float32),
        pltpu.SemaphoreType.DMA((NBUF,)),
        pltpu.SemaphoreType.DMA((NBUF,)),
    ],
    compiler_params=_SC_PARAMS,
)
def _agg_kernel(g_hbm, src_hbm, dst_hbm, out_hbm,
                src_v, dst_v, rows_v, stage_v, acc_sh, gsem, ssem):
    c = lax.axis_index("c")
    s = lax.axis_index("s")
    w = c * NS + s

    _zero_slice_and_barrier(stage_v, acc_sh, s)

    pltpu.sync_copy(src_hbm.at[w], src_v)
    pltpu.sync_copy(dst_hbm.at[w], dst_v)

    def g_desc(j, b):
        return pltpu.make_async_copy(
            g_hbm.at[src_v.at[j]], rows_v.at[b], gsem.at[b])

    def s_desc(j, b):
        return pltpu.make_async_copy(
            rows_v.at[b], acc_sh.at[dst_v.at[j]], ssem.at[b])

    for b in range(NBUF):
        pltpu.async_copy(g_hbm.at[src_v.at[b]], rows_v.at[b], gsem.at[b])

    def group(i, _):
        for b in range(NBUF):
            j = i * NBUF + b
            g_desc(j, b).wait()
            pltpu.async_copy(rows_v.at[b], acc_sh.at[dst_v.at[j]],
                             ssem.at[b], add=True)
        for b in range(NBUF):
            j = i * NBUF + b
            s_desc(j, b).wait()
            jn = j + NBUF
            pltpu.async_copy(g_hbm.at[src_v.at[jn]], rows_v.at[b],
                             gsem.at[b])
        return 0
    lax.fori_loop(0, G - 1, group, 0)

    for b in range(NBUF):
        j = (G - 1) * NBUF + b
        g_desc(j, b).wait()
        pltpu.async_copy(rows_v.at[b], acc_sh.at[dst_v.at[j]],
                         ssem.at[b], add=True)
    for b in range(NBUF):
        s_desc((G - 1) * NBUF + b, b).wait()

    _writeback(stage_v, acc_sh, out_hbm, c, s)


def _tc1_body(degp_ref, x_ref, w1_ref, g1_ref, dinv_ref):
    deg = degp_ref[0] + degp_ref[1] + 1.0
    dinv = lax.rsqrt(deg)
    h = jnp.dot(x_ref[:], w1_ref[:], preferred_element_type=jnp.float32)
    g1_ref[:] = h * dinv
    dinv_ref[:] = dinv


def _tc2_body(accp_ref, g1_ref, dinv_ref, b1_ref, g2_ref):
    dinv = dinv_ref[:]
    z = dinv * (accp_ref[0] + accp_ref[1] + g1_ref[:]) + b1_ref[:]
    g2_ref[:] = jnp.maximum(z, 0.0) * dinv


def _tc3_body(accp_ref, g2_ref, dinv_ref, w2_ref, b2_ref, out_ref):
    z = dinv_ref[:] * (accp_ref[0] + accp_ref[1] + g2_ref[:])
    out_ref[:] = (
        jnp.dot(z, w2_ref[:], preferred_element_type=jnp.float32) + b2_ref[:]
    )


def kernel(x, edge_index, W1, b1, W2, b2):
    src = edge_index[0].reshape(NW, EPW)
    dst = edge_index[1].reshape(NW, EPW)
    src_p = jnp.pad(src, ((0, 0), (0, EPW_P - EPW)),
                    constant_values=N).reshape(NW, C, B)
    dst_p = jnp.pad(dst, ((0, 0), (0, EPW_P - EPW)),
                    constant_values=N).reshape(NW, C, B)
    x_p = jnp.pad(x, ((0, NPAD - N), (0, 0)))
    b1r = b1.reshape(1, HID)
    b2r = b2.reshape(1, OUT_CH)

    deg_parts = _deg_kernel(dst_p)

    g1, dinv16 = pl.pallas_call(
        _tc1_body,
        out_shape=(
            jax.ShapeDtypeStruct((NPAD, HID), jnp.float32),
            jax.ShapeDtypeStruct((NPAD, HID), jnp.float32),
        ),
    )(deg_parts, x_p, W1)

    acc1 = _agg_kernel(g1, src_p, dst_p)

    g2 = pl.pallas_call(
        _tc2_body,
        out_shape=jax.ShapeDtypeStruct((NPAD, HID), jnp.float32),
    )(acc1, g1, dinv16, b1r)

    acc2 = _agg_kernel(g2, src_p, dst_p)

    out = pl.pallas_call(
        _tc3_body,
        out_shape=jax.ShapeDtypeStruct((NPAD, OUT_CH), jnp.float32),
    )(acc2, g2, dinv16, W2, b2r)

    return out[:N]
